# baseline (speedup 1.0000x reference)
import functools

import jax
import jax.numpy as jnp
from jax import lax
from jax.experimental import pallas as pl
from jax.experimental.pallas import tpu as pltpu
from jax.experimental.pallas import tpu_sc as plsc

_THRESHOLD = 0.85

_NC = 2
_NS = 16
_L = 16
_NW = _NC * _NS
_CHUNK = 64
_UNROLL = 8


@functools.lru_cache(maxsize=None)
def _make_sc_kernel(rows: int, k: int):
    rows_per_w = rows // _NW
    n_chunks = rows_per_w // _CHUNK
    mesh = plsc.VectorSubcoreMesh(core_axis_name="c", subcore_axis_name="s")

    @functools.partial(
        pl.kernel,
        mesh=mesh,
        out_type=(
            jax.ShapeDtypeStruct((rows,), jnp.float32),
            jax.ShapeDtypeStruct((rows,), jnp.int32),
        ),
        scratch_types=[
            pltpu.VMEM((_CHUNK, k), jnp.float32),
            pltpu.VMEM((rows_per_w,), jnp.float32),
            pltpu.VMEM((rows_per_w,), jnp.int32),
        ],
        compiler_params=pltpu.CompilerParams(
            use_tc_tiling_on_sc=False, needs_layout_passes=False),
    )
    def sc_kernel(sim_hbm, score_hbm, idx_hbm, buf, acc_s, acc_i):
        wid = lax.axis_index("s") * _NC + lax.axis_index("c")
        base = wid * rows_per_w
        lane = lax.iota(jnp.int32, _L)

        def chunk_body(g, carry):
            pltpu.sync_copy(sim_hbm.at[pl.ds(base + g * _CHUNK, _CHUNK)], buf)
            for r in range(0, _CHUNK, _L):
                row_ids = lane + r

                def step(_, st):
                    vmax, vidx, kvec = st
                    for _u in range(_UNROLL):
                        v = plsc.load_gather(buf, [row_ids, kvec])
                        pred = v > vmax
                        vmax = jnp.where(pred, v, vmax)
                        vidx = jnp.where(pred, kvec, vidx)
                        kvec = kvec + 1
                    return vmax, vidx, kvec

                init = (jnp.full((_L,), -jnp.inf, jnp.float32),
                        jnp.zeros((_L,), jnp.int32),
                        jnp.zeros((_L,), jnp.int32))
                vmax, vidx, _ = lax.fori_loop(0, k // _UNROLL, step, init)
                off = g * _CHUNK + r
                acc_s[pl.ds(off, _L)] = vmax - _THRESHOLD
                acc_i[pl.ds(off, _L)] = vidx
            return carry

        lax.fori_loop(0, n_chunks, chunk_body, 0)
        pltpu.sync_copy(acc_s, score_hbm.at[pl.ds(base, rows_per_w)])
        pltpu.sync_copy(acc_i, idx_hbm.at[pl.ds(base, rows_per_w)])

    return sc_kernel


def kernel(importance, similarity, compressed_map):
    b, n, k = similarity.shape
    rows = b * n
    sim2d = similarity.reshape(rows, k)
    score, idx = _make_sc_kernel(rows, k)(sim2d)
    return (score.reshape(b, n, 1), idx.reshape(b, n))
